# Initial kernel scaffold; baseline (speedup 1.0000x reference)
import functools

import jax
import jax.numpy as jnp
from jax import lax
from jax.experimental import pallas as pl
from jax.experimental.pallas import tpu as pltpu
import jax.experimental.pallas.tpu_sc as plsc

N_NODES = 262144
N_EDGES = 524288
NUM_IN = 128
NUM_HIDDEN = 256
NUM_CLASS = 10
N_GRAPHS = 8192
E_PER_GRAPH = 64

NC = 2
NS = 16
L = 16

_MESH = dict(core_axis_name="c", subcore_axis_name="s", num_cores=NC,
             num_subcores=NS)

_DEG_W = 4096
_E_PER_TILE_DEG = N_EDGES // (NC * NS)
_N_PER_TILE = N_NODES // NS


def _deg_body(index_hbm, ew_win_hbm, degp_hbm, dacc, zbuf, colbuf, ewb):
  c = lax.axis_index("c")
  s = lax.axis_index("s")

  def zero_vec(i, _):
    zbuf[pl.ds(i * L, L)] = jnp.zeros((L,), jnp.float32)
    return 0
  lax.fori_loop(0, _DEG_W // L, zero_vec, 0)

  def zero_chunk(i, _):
    pltpu.sync_copy(zbuf, dacc.at[pl.ds(s * _N_PER_TILE + i * _DEG_W, _DEG_W)])
    return 0
  lax.fori_loop(0, _N_PER_TILE // _DEG_W, zero_chunk, 0)
  plsc.subcore_barrier()

  pltpu.sync_copy(ew_win_hbm, ewb)
  tile_base = (c * NS + s) * _E_PER_TILE_DEG

  def win(w, _):
    ebase = tile_base + w * _DEG_W
    pltpu.sync_copy(index_hbm.at[1, pl.ds(ebase, _DEG_W)], colbuf)
    pltpu.sync_copy(ewb, dacc.at[colbuf], add=True)
    return 0
  lax.fori_loop(0, _E_PER_TILE_DEG // _DEG_W, win, 0)
  plsc.subcore_barrier()

  pltpu.sync_copy(dacc.at[pl.ds(s * _N_PER_TILE, _N_PER_TILE)],
                  degp_hbm.at[c, pl.ds(s * _N_PER_TILE, _N_PER_TILE)])


def _k_deg(index, ew_win):
  f = pl.kernel(
      _deg_body,
      out_type=jax.ShapeDtypeStruct((NC, N_NODES), jnp.float32),
      mesh=plsc.VectorSubcoreMesh(**_MESH),
      scratch_types=[
          pltpu.VMEM_SHARED((N_NODES,), jnp.float32),
          pltpu.VMEM((_DEG_W,), jnp.float32),
          pltpu.VMEM((_DEG_W,), jnp.int32),
          pltpu.VMEM((_DEG_W,), jnp.float32),
      ],
  )
  return f(index, ew_win)


_R = 15360
_CHUNKS = 18
_PASSES = _CHUNKS // NC
_HOP_W = 4096
_E_PER_TILE = N_EDGES // NS
_GK = 512
_LCAP = 4640
_RPT = _R // NS
_TAIL_BASE = 17 * _R
_TAIL_ROWS = N_NODES - _TAIL_BASE
_TAIL_RPT = _TAIL_ROWS // NS


def _hop_fire(src_hbm, acc, flr, flc, flw, st_r, st_c, st_w, gbuf, gsem,
              start):
  def stage(i, _):
    st_r[pl.ds(i * L, L)] = flr[pl.ds(start + i * L, L)]
    st_c[pl.ds(i * L, L)] = flc[pl.ds(start + i * L, L)]
    st_w[pl.ds(i * L, L)] = flw[pl.ds(start + i * L, L)]
    return 0
  lax.fori_loop(0, _GK // L, stage, 0)

  pltpu.async_copy(src_hbm.at[st_r], gbuf, gsem).wait()

  def scale(g, _):
    wv = st_w[pl.ds(g * L, L)]
    for k in range(L):
      wsp = jnp.full((L,), wv[k], jnp.float32)
      r = g * L + k
      for q in range(NUM_IN // L):
        gbuf[r, pl.ds(q * L, L)] = gbuf[r, pl.ds(q * L, L)] * wsp
    return 0
  lax.fori_loop(0, _GK // L, scale, 0)

  pltpu.sync_copy(gbuf, acc.at[st_c], add=True)


def _hop_body(src_hbm, index_hbm, w64_hbm, dst_hbm,
              acc, flr, flc, flw, st_r, st_c, st_w, gbuf, colw, roww, ewb,
              gsem):
  c = lax.axis_index("c")
  s = lax.axis_index("s")
  pltpu.sync_copy(w64_hbm, ewb)
  lane = lax.iota(jnp.int32, (L,))

  def do_pass(p, _):
    k = NC * p + c
    is_tail = k == (_CHUNKS - 1)
    base = jnp.where(is_tail, _TAIL_BASE, k * _R)
    init_base = jnp.where(is_tail, N_NODES - _R, k * _R)
    hi = jnp.where(is_tail, N_NODES, k * _R + _R)

    pltpu.sync_copy(src_hbm.at[pl.ds(init_base + s * _RPT, _RPT)],
                    acc.at[pl.ds(s * _RPT, _RPT)])
    plsc.subcore_barrier()

    def drain(cnt, fired):
      def one(d, fired):
        @pl.when(fired + _GK <= cnt)
        def _():
          _hop_fire(src_hbm, acc, flr, flc, flw, st_r, st_c, st_w, gbuf,
                    gsem, fired)
        return fired + jnp.where(fired + _GK <= cnt, _GK, 0)
      fired = lax.fori_loop(0, (_LCAP + _GK - 1) // _GK, one, fired)
      rem = cnt - fired
      def mv(i, _):
        @pl.when(i * L < rem)
        def _():
          flr[pl.ds(i * L, L)] = flr[pl.ds(fired + i * L, L)]
          flc[pl.ds(i * L, L)] = flc[pl.ds(fired + i * L, L)]
          flw[pl.ds(i * L, L)] = flw[pl.ds(fired + i * L, L)]
        return 0
      lax.fori_loop(0, _GK // L, mv, 0)
      return rem

    def win(w, cnt):
      ebase = s * _E_PER_TILE + w * _HOP_W
      pltpu.sync_copy(index_hbm.at[1, pl.ds(ebase, _HOP_W)], colw)
      pltpu.sync_copy(index_hbm.at[0, pl.ds(ebase, _HOP_W)], roww)

      def scan(j, cnt):
        c16 = colw[pl.ds(j * L, L)]
        m = (c16 >= base) & (c16 < hi)
        r16 = roww[pl.ds(j * L, L)]
        w16 = ewb[pl.ds(lax.rem(j, 4) * L, L)]
        lc16 = c16 - init_base
        plsc.store_compressed(flr.at[pl.ds(cnt, L)], r16, mask=m)
        plsc.store_compressed(flc.at[pl.ds(cnt, L)], lc16, mask=m)
        plsc.store_compressed(flw.at[pl.ds(cnt, L)], w16, mask=m)
        pc = plsc.all_reduce_population_count(m)
        return cnt + pc[0]
      cnt = lax.fori_loop(0, _HOP_W // L, scan, cnt)
      return drain(cnt, 0)

    cnt = lax.fori_loop(0, _E_PER_TILE // _HOP_W, win, 0)

    cnt_pad = jnp.where(cnt > 0, ((cnt + _GK - 1) // _GK) * _GK, 0)
    def pad(i, _):
      @pl.when(cnt + i * L < cnt_pad)
      def _():
        off = cnt + i * L
        flr[pl.ds(off, L)] = lane
        flc[pl.ds(off, L)] = lane
        flw[pl.ds(off, L)] = jnp.zeros((L,), jnp.float32)
      return 0
    lax.fori_loop(0, _GK // L, pad, 0)
    drain(cnt_pad, 0)

    plsc.subcore_barrier()
    @pl.when(jnp.logical_not(is_tail))
    def _():
      pltpu.sync_copy(acc.at[pl.ds(s * _RPT, _RPT)],
                      dst_hbm.at[pl.ds(init_base + s * _RPT, _RPT)])
    @pl.when(is_tail)
    def _():
      pltpu.sync_copy(
          acc.at[pl.ds(_R - _TAIL_ROWS + s * _TAIL_RPT, _TAIL_RPT)],
          dst_hbm.at[pl.ds(_TAIL_BASE + s * _TAIL_RPT, _TAIL_RPT)])
    plsc.subcore_barrier()
    return 0

  lax.fori_loop(0, _PASSES, do_pass, 0)


def _k_hop(src, index, w64):
  f = pl.kernel(
      _hop_body,
      out_type=jax.ShapeDtypeStruct((N_NODES, NUM_IN), jnp.float32),
      mesh=plsc.VectorSubcoreMesh(**_MESH),
      scratch_types=[
          pltpu.VMEM_SHARED((_R, NUM_IN), jnp.float32),
          pltpu.VMEM((_LCAP,), jnp.int32),
          pltpu.VMEM((_LCAP,), jnp.int32),
          pltpu.VMEM((_LCAP,), jnp.float32),
          pltpu.VMEM((_GK,), jnp.int32),
          pltpu.VMEM((_GK,), jnp.int32),
          pltpu.VMEM((_GK,), jnp.float32),
          pltpu.VMEM((_GK, NUM_IN), jnp.float32),
          pltpu.VMEM((_HOP_W,), jnp.int32),
          pltpu.VMEM((_HOP_W,), jnp.int32),
          pltpu.VMEM((E_PER_GRAPH,), jnp.float32),
          pltpu.SemaphoreType.DMA,
      ],
  )
  return f(src, index, w64)


_POOL_W = 2048
_ROWS_PER_TILE = N_NODES // (NC * NS)
_G_PER_TILE = N_GRAPHS // NS
LG = 16


def _pool_body(lg_hbm, batch_hbm, out_hbm, pacc, zbuf, rbuf, bbuf):
  c = lax.axis_index("c")
  s = lax.axis_index("s")

  def zero_vec(i, _):
    zbuf[i, :] = jnp.zeros((L,), jnp.float32)
    return 0
  lax.fori_loop(0, _G_PER_TILE, zero_vec, 0)
  pltpu.sync_copy(zbuf, pacc.at[pl.ds(s * _G_PER_TILE, _G_PER_TILE)])
  plsc.subcore_barrier()

  tile_base = (c * NS + s) * _ROWS_PER_TILE

  def win(w, _):
    rbase = tile_base + w * _POOL_W
    pltpu.sync_copy(lg_hbm.at[pl.ds(rbase, _POOL_W)], rbuf)
    pltpu.sync_copy(batch_hbm.at[pl.ds(rbase, _POOL_W)], bbuf)
    pltpu.sync_copy(rbuf, pacc.at[bbuf], add=True)
    return 0
  lax.fori_loop(0, _ROWS_PER_TILE // _POOL_W, win, 0)
  plsc.subcore_barrier()

  pltpu.sync_copy(pacc.at[pl.ds(s * _G_PER_TILE, _G_PER_TILE)],
                  out_hbm.at[c, pl.ds(s * _G_PER_TILE, _G_PER_TILE)])


def _k_pool(lg, batch):
  f = pl.kernel(
      _pool_body,
      out_type=jax.ShapeDtypeStruct((NC, N_GRAPHS, LG), jnp.float32),
      mesh=plsc.VectorSubcoreMesh(**_MESH),
      scratch_types=[
          pltpu.VMEM_SHARED((N_GRAPHS, LG), jnp.float32),
          pltpu.VMEM((_G_PER_TILE, LG), jnp.float32),
          pltpu.VMEM((_POOL_W, LG), jnp.float32),
          pltpu.VMEM((_POOL_W,), jnp.int32),
      ],
  )
  return f(lg, batch)


_PRE_B = 8192


def _pre_body(degp_ref, x_ref, dinv_ref, dinv2_ref, y0_ref):
  deg = 1.0 + degp_ref[0] + degp_ref[1]
  dinv = lax.rsqrt(deg)
  dinv_ref[...] = dinv
  dinv2_ref[...] = 1.0 / deg
  y0_ref[...] = x_ref[...] * dinv


def _k_pre(degp, x):
  grid = N_NODES // _PRE_B
  return pl.pallas_call(
      _pre_body,
      grid=(grid,),
      in_specs=[
          pl.BlockSpec((NC, _PRE_B, 1), lambda i: (0, i, 0)),
          pl.BlockSpec((_PRE_B, NUM_IN), lambda i: (i, 0)),
      ],
      out_specs=[
          pl.BlockSpec((_PRE_B, 1), lambda i: (i, 0)),
          pl.BlockSpec((_PRE_B, 1), lambda i: (i, 0)),
          pl.BlockSpec((_PRE_B, NUM_IN), lambda i: (i, 0)),
      ],
      out_shape=[
          jax.ShapeDtypeStruct((N_NODES, 1), jnp.float32),
          jax.ShapeDtypeStruct((N_NODES, 1), jnp.float32),
          jax.ShapeDtypeStruct((N_NODES, NUM_IN), jnp.float32),
      ],
  )(degp.reshape(NC, N_NODES, 1), x)


def _scale_body(y_ref, d_ref, o_ref):
  o_ref[...] = y_ref[...] * d_ref[...]


def _k_scale(y, d):
  grid = N_NODES // _PRE_B
  return pl.pallas_call(
      _scale_body,
      grid=(grid,),
      in_specs=[
          pl.BlockSpec((_PRE_B, NUM_IN), lambda i: (i, 0)),
          pl.BlockSpec((_PRE_B, 1), lambda i: (i, 0)),
      ],
      out_specs=pl.BlockSpec((_PRE_B, NUM_IN), lambda i: (i, 0)),
      out_shape=jax.ShapeDtypeStruct((N_NODES, NUM_IN), jnp.float32),
  )(y, d)


_MM_B = 2048


def _mm_body(y3_ref, dinv_ref, linwT_ref, linb_ref, fcwT_ref, o_ref):
  h2 = y3_ref[...] * dinv_ref[...]
  h = jnp.dot(h2, linwT_ref[...], preferred_element_type=jnp.float32)
  h = jnp.maximum(h + linb_ref[...], 0.0)
  o_ref[...] = jnp.dot(h, fcwT_ref[...], preferred_element_type=jnp.float32)


def _k_mm(y3, dinv, linwT, linb, fcwT):
  grid = N_NODES // _MM_B
  return pl.pallas_call(
      _mm_body,
      grid=(grid,),
      in_specs=[
          pl.BlockSpec((_MM_B, NUM_IN), lambda i: (i, 0)),
          pl.BlockSpec((_MM_B, 1), lambda i: (i, 0)),
          pl.BlockSpec((NUM_IN, NUM_HIDDEN), lambda i: (0, 0)),
          pl.BlockSpec((1, NUM_HIDDEN), lambda i: (0, 0)),
          pl.BlockSpec((NUM_HIDDEN, LG), lambda i: (0, 0)),
      ],
      out_specs=pl.BlockSpec((_MM_B, LG), lambda i: (i, 0)),
      out_shape=jax.ShapeDtypeStruct((N_NODES, LG), jnp.float32),
  )(y3, dinv, linwT, linb, fcwT)


def _soft_body(pp_ref, fcb_ref, o_ref):
  z = pp_ref[0] + pp_ref[1] + fcb_ref[...]
  col = lax.broadcasted_iota(jnp.int32, (N_GRAPHS, LG), 1)
  valid = col < NUM_CLASS
  z = jnp.where(valid, z, -1e30)
  z = z - jnp.max(z, axis=1, keepdims=True)
  p = jnp.exp(z)
  p = jnp.where(valid, p, 0.0)
  o_ref[...] = p / jnp.sum(p, axis=1, keepdims=True)


def _k_soft(pooledp, fcb):
  return pl.pallas_call(
      _soft_body,
      in_specs=[
          pl.BlockSpec((NC, N_GRAPHS, LG), lambda: (0, 0, 0)),
          pl.BlockSpec((1, LG), lambda: (0, 0)),
      ],
      out_specs=pl.BlockSpec((N_GRAPHS, LG), lambda: (0, 0)),
      out_shape=jax.ShapeDtypeStruct((N_GRAPHS, LG), jnp.float32),
  )(pooledp, fcb)


def kernel(x, index, batch, weight, lin_w, lin_b, fc_w, fc_b):
  index = index.astype(jnp.int32)
  batch = batch.astype(jnp.int32)
  ew_win = jnp.tile(weight, _DEG_W // E_PER_GRAPH)

  degp = _k_deg(index, ew_win)
  dinv, dinv2, y0 = _k_pre(degp, x)
  y1 = _k_hop(y0, index, weight)
  y2 = _k_scale(y1, dinv2)
  y3 = _k_hop(y2, index, weight)

  linwT = lin_w.T
  linb = lin_b.reshape(1, NUM_HIDDEN)
  fcwT = jnp.zeros((NUM_HIDDEN, LG), jnp.float32).at[:, :NUM_CLASS].set(fc_w.T)
  lg = _k_mm(y3, dinv, linwT, linb, fcwT)

  pooledp = _k_pool(lg, batch)
  fcb = jnp.zeros((1, LG), jnp.float32).at[0, :NUM_CLASS].set(fc_b)
  probs = _k_soft(pooledp, fcb)
  return probs[:, :NUM_CLASS]


# speedup vs baseline: 7.3078x; 7.3078x over previous
import functools

import jax
import jax.numpy as jnp
from jax import lax
from jax.experimental import pallas as pl
from jax.experimental.pallas import tpu as pltpu
import jax.experimental.pallas.tpu_sc as plsc

N_NODES = 262144
N_EDGES = 524288
NUM_IN = 128
NUM_HIDDEN = 256
NUM_CLASS = 10
N_GRAPHS = 8192
E_PER_GRAPH = 64

NC = 2
NS = 16
L = 16

_MESH = dict(core_axis_name="c", subcore_axis_name="s", num_cores=NC,
             num_subcores=NS)

_DEG_W = 4096
_E_PER_TILE_DEG = N_EDGES // (NC * NS)
_N_PER_TILE = N_NODES // NS


def _deg_body(index_hbm, ew_win_hbm, degp_hbm, dacc, zbuf, colbuf, ewb):
  c = lax.axis_index("c")
  s = lax.axis_index("s")

  def zero_vec(i, _):
    zbuf[pl.ds(i * L, L)] = jnp.zeros((L,), jnp.float32)
    return 0
  lax.fori_loop(0, _DEG_W // L, zero_vec, 0)

  def zero_chunk(i, _):
    pltpu.sync_copy(zbuf, dacc.at[pl.ds(s * _N_PER_TILE + i * _DEG_W, _DEG_W)])
    return 0
  lax.fori_loop(0, _N_PER_TILE // _DEG_W, zero_chunk, 0)
  plsc.subcore_barrier()

  pltpu.sync_copy(ew_win_hbm, ewb)
  tile_base = (c * NS + s) * _E_PER_TILE_DEG

  def win(w, _):
    ebase = tile_base + w * _DEG_W
    pltpu.sync_copy(index_hbm.at[1, pl.ds(ebase, _DEG_W)], colbuf)
    pltpu.sync_copy(ewb, dacc.at[colbuf], add=True)
    return 0
  lax.fori_loop(0, _E_PER_TILE_DEG // _DEG_W, win, 0)
  plsc.subcore_barrier()

  pltpu.sync_copy(dacc.at[pl.ds(s * _N_PER_TILE, _N_PER_TILE)],
                  degp_hbm.at[c, pl.ds(s * _N_PER_TILE, _N_PER_TILE)])


def _k_deg(index, ew_win):
  f = pl.kernel(
      _deg_body,
      out_type=jax.ShapeDtypeStruct((NC, N_NODES), jnp.float32),
      mesh=plsc.VectorSubcoreMesh(**_MESH),
      compiler_params=pltpu.CompilerParams(needs_layout_passes=False),
      scratch_types=[
          pltpu.VMEM_SHARED((N_NODES,), jnp.float32),
          pltpu.VMEM((_DEG_W,), jnp.float32),
          pltpu.VMEM((_DEG_W,), jnp.int32),
          pltpu.VMEM((_DEG_W,), jnp.float32),
      ],
  )
  return f(index, ew_win)


_R = 12160
_CHUNKS = 22
_PASSES = _CHUNKS // NC
_HOP_W = 2048
_E_PER_TILE = N_EDGES // NS
_GK = 128
_LCAP = 2608
_RPT = _R // NS
_TAIL_BASE = (_CHUNKS - 1) * _R
_TAIL_ROWS = N_NODES - _TAIL_BASE
_TAIL_RPT = _TAIL_ROWS // NS


def _hop_fire(src_hbm, acc, flr, flc, flw, st_r, st_c, st_w, gbuf, gsem,
              start):
  def stage(i, _):
    st_r[pl.ds(i * L, L)] = flr[pl.ds(start + i * L, L)]
    st_c[pl.ds(i * L, L)] = flc[pl.ds(start + i * L, L)]
    st_w[pl.ds(i * L, L)] = flw[pl.ds(start + i * L, L)]
    return 0
  lax.fori_loop(0, _GK // L, stage, 0)

  pltpu.async_copy(src_hbm.at[st_r], gbuf, gsem).wait()

  def scale(g, _):
    wv = st_w[pl.ds(g * L, L)]
    for k in range(L):
      wsp = jnp.full((L,), wv[k], jnp.float32)
      r = g * L + k
      for q in range(NUM_IN // L):
        gbuf[r, pl.ds(q * L, L)] = gbuf[r, pl.ds(q * L, L)] * wsp
    return 0
  lax.fori_loop(0, _GK // L, scale, 0)

  pltpu.sync_copy(gbuf, acc.at[st_c], add=True)


def _hop_body(src_hbm, index_hbm, w64_hbm, dst_hbm,
              acc, flr, flc, flw, st_r, st_c, st_w, gbuf, colw, roww, ewb,
              gsem):
  c = lax.axis_index("c")
  s = lax.axis_index("s")
  pltpu.sync_copy(w64_hbm, ewb)
  lane = lax.iota(jnp.int32, L)

  def do_pass(p, _):
    k = NC * p + c
    is_tail = k == (_CHUNKS - 1)
    base = jnp.where(is_tail, _TAIL_BASE, k * _R)
    init_base = jnp.where(is_tail, N_NODES - _R, k * _R)
    hi = jnp.where(is_tail, N_NODES, k * _R + _R)

    pltpu.sync_copy(src_hbm.at[pl.ds(init_base + s * _RPT, _RPT)],
                    acc.at[pl.ds(s * _RPT, _RPT)])
    plsc.subcore_barrier()

    def drain(cnt, fired):
      def one(d, fired):
        @pl.when(fired + _GK <= cnt)
        def _():
          _hop_fire(src_hbm, acc, flr, flc, flw, st_r, st_c, st_w, gbuf,
                    gsem, fired)
        return fired + jnp.where(fired + _GK <= cnt, _GK, 0)
      fired = lax.fori_loop(0, (_LCAP + _GK - 1) // _GK, one, fired)
      rem = cnt - fired
      def mv(i, _):
        @pl.when(i * L < rem)
        def _():
          flr[pl.ds(i * L, L)] = flr[pl.ds(fired + i * L, L)]
          flc[pl.ds(i * L, L)] = flc[pl.ds(fired + i * L, L)]
          flw[pl.ds(i * L, L)] = flw[pl.ds(fired + i * L, L)]
        return 0
      lax.fori_loop(0, _GK // L, mv, 0)
      return rem

    def win(w, cnt):
      ebase = s * _E_PER_TILE + w * _HOP_W
      pltpu.sync_copy(index_hbm.at[1, pl.ds(ebase, _HOP_W)], colw)
      pltpu.sync_copy(index_hbm.at[0, pl.ds(ebase, _HOP_W)], roww)

      def scan(j, cnt):
        c16 = colw[pl.ds(j * L, L)]
        m = (c16 >= base) & (c16 < hi)
        r16 = roww[pl.ds(j * L, L)]
        w16 = ewb[pl.ds(lax.rem(j, 4) * L, L)]
        lc16 = c16 - init_base
        cs = plsc.cumsum(m.astype(jnp.int32))
        pos = cs + (cnt - 1)
        plsc.store_scatter(flr, [pos], r16, mask=m)
        plsc.store_scatter(flc, [pos], lc16, mask=m)
        plsc.store_scatter(flw, [pos], w16, mask=m)
        return cnt + cs[L - 1]
      cnt = lax.fori_loop(0, _HOP_W // L, scan, cnt)
      return drain(cnt, 0)

    cnt = lax.fori_loop(0, _E_PER_TILE // _HOP_W, win, 0)

    cnt_pad = jnp.where(cnt > 0, ((cnt + _GK - 1) // _GK) * _GK, 0)
    def pad(i, _):
      @pl.when(cnt + i * L < cnt_pad)
      def _():
        off = cnt + i * L
        flr[pl.ds(off, L)] = lane
        flc[pl.ds(off, L)] = lane
        flw[pl.ds(off, L)] = jnp.zeros((L,), jnp.float32)
      return 0
    lax.fori_loop(0, _GK // L, pad, 0)
    drain(cnt_pad, 0)

    plsc.subcore_barrier()
    @pl.when(jnp.logical_not(is_tail))
    def _():
      pltpu.sync_copy(acc.at[pl.ds(s * _RPT, _RPT)],
                      dst_hbm.at[pl.ds(init_base + s * _RPT, _RPT)])
    @pl.when(is_tail)
    def _():
      pltpu.sync_copy(
          acc.at[pl.ds(_R - _TAIL_ROWS + s * _TAIL_RPT, _TAIL_RPT)],
          dst_hbm.at[pl.ds(_TAIL_BASE + s * _TAIL_RPT, _TAIL_RPT)])
    plsc.subcore_barrier()
    return 0

  lax.fori_loop(0, _PASSES, do_pass, 0)


def _k_hop(src, index, w64):
  f = pl.kernel(
      _hop_body,
      out_type=jax.ShapeDtypeStruct((N_NODES, NUM_IN), jnp.float32),
      mesh=plsc.VectorSubcoreMesh(**_MESH),
      compiler_params=pltpu.CompilerParams(needs_layout_passes=False),
      scratch_types=[
          pltpu.VMEM_SHARED((_R, NUM_IN), jnp.float32),
          pltpu.VMEM((_LCAP,), jnp.int32),
          pltpu.VMEM((_LCAP,), jnp.int32),
          pltpu.VMEM((_LCAP,), jnp.float32),
          pltpu.VMEM((_GK,), jnp.int32),
          pltpu.VMEM((_GK,), jnp.int32),
          pltpu.VMEM((_GK,), jnp.float32),
          pltpu.VMEM((_GK, NUM_IN), jnp.float32),
          pltpu.VMEM((_HOP_W,), jnp.int32),
          pltpu.VMEM((_HOP_W,), jnp.int32),
          pltpu.VMEM((E_PER_GRAPH,), jnp.float32),
          pltpu.SemaphoreType.DMA,
      ],
  )
  return f(src, index, w64)


_POOL_W = 2048
_ROWS_PER_TILE = N_NODES // (NC * NS)
_G_PER_TILE = N_GRAPHS // NS
LG = 16


_PACC = N_GRAPHS * LG
_ZP = _PACC // NS


def _pool_body(lg_hbm, batch_hbm, out_hbm, pacc, zbuf, rbuf, bbuf, ibuf):
  c = lax.axis_index("c")
  s = lax.axis_index("s")
  lane = lax.iota(jnp.int32, L)

  def zero_vec(i, _):
    zbuf[pl.ds(i * L, L)] = jnp.zeros((L,), jnp.float32)
    return 0
  lax.fori_loop(0, _ZP // L, zero_vec, 0)
  pltpu.sync_copy(zbuf, pacc.at[pl.ds(s * _ZP, _ZP)])
  plsc.subcore_barrier()

  tile_base = (c * NS + s) * _ROWS_PER_TILE

  def win(w, _):
    rbase = tile_base + w * _POOL_W
    pltpu.sync_copy(lg_hbm.at[pl.ds(rbase * LG, _POOL_W * LG)], rbuf)
    pltpu.sync_copy(batch_hbm.at[pl.ds(rbase, _POOL_W)], bbuf)

    mall = lane >= 0
    def expand(g, _):
      bv = bbuf[pl.ds(g * L, L)] * LG
      ppos = g * (L * LG) + lane * LG
      for j in range(LG):
        plsc.store_scatter(ibuf, [ppos + j], bv + j, mask=mall)
      return 0
    lax.fori_loop(0, _POOL_W // L, expand, 0)
    pltpu.sync_copy(rbuf, pacc.at[ibuf], add=True)
    return 0
  lax.fori_loop(0, _ROWS_PER_TILE // _POOL_W, win, 0)
  plsc.subcore_barrier()

  pltpu.sync_copy(pacc.at[pl.ds(s * _ZP, _ZP)],
                  out_hbm.at[c, pl.ds(s * _ZP, _ZP)])


def _k_pool(lg_flat, batch):
  f = pl.kernel(
      _pool_body,
      out_type=jax.ShapeDtypeStruct((NC, _PACC), jnp.float32),
      mesh=plsc.VectorSubcoreMesh(**_MESH),
      compiler_params=pltpu.CompilerParams(needs_layout_passes=False),
      scratch_types=[
          pltpu.VMEM_SHARED((_PACC,), jnp.float32),
          pltpu.VMEM((_ZP,), jnp.float32),
          pltpu.VMEM((_POOL_W * LG,), jnp.float32),
          pltpu.VMEM((_POOL_W,), jnp.int32),
          pltpu.VMEM((_POOL_W * LG,), jnp.int32),
      ],
  )
  return f(lg_flat, batch)


_PRE_B = 1024


def _pre_body(degp_ref, x_ref, dinv_ref, dinv2_ref, y0_ref):
  deg = 1.0 + degp_ref[0] + degp_ref[1]
  dinv = lax.rsqrt(deg)
  dinv_ref[...] = dinv
  dinv2_ref[...] = 1.0 / deg
  y0_ref[...] = x_ref[...] * dinv


def _k_pre(degp, x):
  grid = N_NODES // _PRE_B
  return pl.pallas_call(
      _pre_body,
      grid=(grid,),
      in_specs=[
          pl.BlockSpec((NC, _PRE_B, 1), lambda i: (0, i, 0)),
          pl.BlockSpec((_PRE_B, NUM_IN), lambda i: (i, 0)),
      ],
      out_specs=[
          pl.BlockSpec((_PRE_B, 1), lambda i: (i, 0)),
          pl.BlockSpec((_PRE_B, 1), lambda i: (i, 0)),
          pl.BlockSpec((_PRE_B, NUM_IN), lambda i: (i, 0)),
      ],
      out_shape=[
          jax.ShapeDtypeStruct((N_NODES, 1), jnp.float32),
          jax.ShapeDtypeStruct((N_NODES, 1), jnp.float32),
          jax.ShapeDtypeStruct((N_NODES, NUM_IN), jnp.float32),
      ],
  )(degp.reshape(NC, N_NODES, 1), x)


def _scale_body(y_ref, d_ref, o_ref):
  o_ref[...] = y_ref[...] * d_ref[...]


def _k_scale(y, d):
  grid = N_NODES // _PRE_B
  return pl.pallas_call(
      _scale_body,
      grid=(grid,),
      in_specs=[
          pl.BlockSpec((_PRE_B, NUM_IN), lambda i: (i, 0)),
          pl.BlockSpec((_PRE_B, 1), lambda i: (i, 0)),
      ],
      out_specs=pl.BlockSpec((_PRE_B, NUM_IN), lambda i: (i, 0)),
      out_shape=jax.ShapeDtypeStruct((N_NODES, NUM_IN), jnp.float32),
  )(y, d)


_MM_B = 2048


def _mm_body(y3_ref, dinv_ref, linwT_ref, linb_ref, fcwT_ref, o_ref):
  h2 = y3_ref[...] * dinv_ref[...]
  h = jnp.dot(h2, linwT_ref[...], preferred_element_type=jnp.float32)
  h = jnp.maximum(h + linb_ref[...], 0.0)
  o_ref[...] = jnp.dot(h, fcwT_ref[...], preferred_element_type=jnp.float32)


def _k_mm(y3, dinv, linwT, linb, fcwT):
  grid = N_NODES // _MM_B
  return pl.pallas_call(
      _mm_body,
      grid=(grid,),
      in_specs=[
          pl.BlockSpec((_MM_B, NUM_IN), lambda i: (i, 0)),
          pl.BlockSpec((_MM_B, 1), lambda i: (i, 0)),
          pl.BlockSpec((NUM_IN, NUM_HIDDEN), lambda i: (0, 0)),
          pl.BlockSpec((1, NUM_HIDDEN), lambda i: (0, 0)),
          pl.BlockSpec((NUM_HIDDEN, LG), lambda i: (0, 0)),
      ],
      out_specs=pl.BlockSpec((_MM_B, LG), lambda i: (i, 0)),
      out_shape=jax.ShapeDtypeStruct((N_NODES, LG), jnp.float32),
  )(y3, dinv, linwT, linb, fcwT)


def _soft_body(pp_ref, fcb_ref, o_ref):
  z = pp_ref[0] + pp_ref[1] + fcb_ref[...]
  col = lax.broadcasted_iota(jnp.int32, (N_GRAPHS, LG), 1)
  valid = col < NUM_CLASS
  z = jnp.where(valid, z, -1e30)
  z = z - jnp.max(z, axis=1, keepdims=True)
  p = jnp.exp(z)
  p = jnp.where(valid, p, 0.0)
  o_ref[...] = p / jnp.sum(p, axis=1, keepdims=True)


def _k_soft(pooledp, fcb):
  return pl.pallas_call(
      _soft_body,
      in_specs=[
          pl.BlockSpec((NC, N_GRAPHS, LG), lambda: (0, 0, 0)),
          pl.BlockSpec((1, LG), lambda: (0, 0)),
      ],
      out_specs=pl.BlockSpec((N_GRAPHS, LG), lambda: (0, 0)),
      out_shape=jax.ShapeDtypeStruct((N_GRAPHS, LG), jnp.float32),
  )(pooledp, fcb)


def kernel(x, index, batch, weight, lin_w, lin_b, fc_w, fc_b):
  index = index.astype(jnp.int32)
  batch = batch.astype(jnp.int32)
  ew_win = jnp.tile(weight, _DEG_W // E_PER_GRAPH)

  degp = _k_deg(index, ew_win)
  dinv, dinv2, y0 = _k_pre(degp, x)
  y1 = _k_hop(y0, index, weight)
  y2 = _k_scale(y1, dinv2)
  y3 = _k_hop(y2, index, weight)

  linwT = lin_w.T
  linb = lin_b.reshape(1, NUM_HIDDEN)
  fcwT = jnp.zeros((NUM_HIDDEN, LG), jnp.float32).at[:, :NUM_CLASS].set(fc_w.T)
  lg = _k_mm(y3, dinv, linwT, linb, fcwT)

  pooledp = _k_pool(lg.reshape(-1), batch)
  fcb = jnp.zeros((1, LG), jnp.float32).at[0, :NUM_CLASS].set(fc_b)
  probs = _k_soft(pooledp.reshape(NC, N_GRAPHS, LG), fcb)
  return probs[:, :NUM_CLASS]


# speedup vs baseline: 8.2920x; 1.1347x over previous
import functools

import jax
import jax.numpy as jnp
from jax import lax
from jax.experimental import pallas as pl
from jax.experimental.pallas import tpu as pltpu
import jax.experimental.pallas.tpu_sc as plsc

N_NODES = 262144
N_EDGES = 524288
NUM_IN = 128
NUM_HIDDEN = 256
NUM_CLASS = 10
N_GRAPHS = 8192
E_PER_GRAPH = 64

NC = 2
NS = 16
L = 16

_MESH = dict(core_axis_name="c", subcore_axis_name="s", num_cores=NC,
             num_subcores=NS)

_DEG_W = 4096
_E_PER_TILE_DEG = N_EDGES // (NC * NS)
_N_PER_TILE = N_NODES // NS


def _deg_body(index_hbm, ew_win_hbm, degp_hbm, dacc, zbuf, colbuf, ewb):
  c = lax.axis_index("c")
  s = lax.axis_index("s")

  def zero_vec(i, _):
    zbuf[pl.ds(i * L, L)] = jnp.zeros((L,), jnp.float32)
    return 0
  lax.fori_loop(0, _DEG_W // L, zero_vec, 0)

  def zero_chunk(i, _):
    pltpu.sync_copy(zbuf, dacc.at[pl.ds(s * _N_PER_TILE + i * _DEG_W, _DEG_W)])
    return 0
  lax.fori_loop(0, _N_PER_TILE // _DEG_W, zero_chunk, 0)
  plsc.subcore_barrier()

  pltpu.sync_copy(ew_win_hbm, ewb)
  tile_base = (c * NS + s) * _E_PER_TILE_DEG

  def win(w, _):
    ebase = tile_base + w * _DEG_W
    pltpu.sync_copy(index_hbm.at[1, pl.ds(ebase, _DEG_W)], colbuf)
    pltpu.sync_copy(ewb, dacc.at[colbuf], add=True)
    return 0
  lax.fori_loop(0, _E_PER_TILE_DEG // _DEG_W, win, 0)
  plsc.subcore_barrier()

  pltpu.sync_copy(dacc.at[pl.ds(s * _N_PER_TILE, _N_PER_TILE)],
                  degp_hbm.at[c, pl.ds(s * _N_PER_TILE, _N_PER_TILE)])


def _k_deg(index, ew_win):
  f = pl.kernel(
      _deg_body,
      out_type=jax.ShapeDtypeStruct((NC, N_NODES), jnp.float32),
      mesh=plsc.VectorSubcoreMesh(**_MESH),
      compiler_params=pltpu.CompilerParams(needs_layout_passes=False),
      scratch_types=[
          pltpu.VMEM_SHARED((N_NODES,), jnp.float32),
          pltpu.VMEM((_DEG_W,), jnp.float32),
          pltpu.VMEM((_DEG_W,), jnp.int32),
          pltpu.VMEM((_DEG_W,), jnp.float32),
      ],
  )
  return f(index, ew_win)


_R = 12160
_CHUNKS = 22
_PASSES = _CHUNKS // NC
_HOP_W = 2048
_E_PER_TILE = N_EDGES // NS
_GK = 128
_LCAP = 2608
_RPT = _R // NS
_TAIL_BASE = (_CHUNKS - 1) * _R
_TAIL_ROWS = N_NODES - _TAIL_BASE
_TAIL_RPT = _TAIL_ROWS // NS


def _hop_fire(src_hbm, acc, flr, flc, flw, st_r, st_c, st_w, gbuf, gsem,
              start):
  def stage(i, _):
    st_r[pl.ds(i * L, L)] = flr[pl.ds(start + i * L, L)]
    st_c[pl.ds(i * L, L)] = flc[pl.ds(start + i * L, L)]
    st_w[pl.ds(i * L, L)] = flw[pl.ds(start + i * L, L)]
    return 0
  lax.fori_loop(0, _GK // L, stage, 0)

  pltpu.async_copy(src_hbm.at[st_r], gbuf, gsem).wait()

  def scale(g, _):
    wv = st_w[pl.ds(g * L, L)]
    for k in range(L):
      wsp = jnp.full((L,), wv[k], jnp.float32)
      r = g * L + k
      for q in range(NUM_IN // L):
        gbuf[r, pl.ds(q * L, L)] = gbuf[r, pl.ds(q * L, L)] * wsp
    return 0
  lax.fori_loop(0, _GK // L, scale, 0)

  pltpu.sync_copy(gbuf, acc.at[st_c], add=True)


def _hop_body(src_hbm, index_hbm, w64_hbm, dst_hbm,
              acc, flr, flc, flw, st_r, st_c, st_w, gbuf, colw, roww, ewb,
              gsem):
  c = lax.axis_index("c")
  s = lax.axis_index("s")
  pltpu.sync_copy(w64_hbm, ewb)
  lane = lax.iota(jnp.int32, L)

  def do_pass(p, _):
    k = NC * p + c
    is_tail = k == (_CHUNKS - 1)
    base = jnp.where(is_tail, _TAIL_BASE, k * _R)
    init_base = jnp.where(is_tail, N_NODES - _R, k * _R)
    hi = jnp.where(is_tail, N_NODES, k * _R + _R)

    pltpu.sync_copy(src_hbm.at[pl.ds(init_base + s * _RPT, _RPT)],
                    acc.at[pl.ds(s * _RPT, _RPT)])
    plsc.subcore_barrier()

    def drain(cnt, fired):
      def one(d, fired):
        @pl.when(fired + _GK <= cnt)
        def _():
          _hop_fire(src_hbm, acc, flr, flc, flw, st_r, st_c, st_w, gbuf,
                    gsem, fired)
        return fired + jnp.where(fired + _GK <= cnt, _GK, 0)
      fired = lax.fori_loop(0, (_LCAP + _GK - 1) // _GK, one, fired)
      rem = cnt - fired
      def mv(i, _):
        @pl.when(i * L < rem)
        def _():
          flr[pl.ds(i * L, L)] = flr[pl.ds(fired + i * L, L)]
          flc[pl.ds(i * L, L)] = flc[pl.ds(fired + i * L, L)]
          flw[pl.ds(i * L, L)] = flw[pl.ds(fired + i * L, L)]
        return 0
      lax.fori_loop(0, _GK // L, mv, 0)
      return rem

    def win(w, cnt):
      ebase = s * _E_PER_TILE + w * _HOP_W
      pltpu.sync_copy(index_hbm.at[1, pl.ds(ebase, _HOP_W)], colw)
      pltpu.sync_copy(index_hbm.at[0, pl.ds(ebase, _HOP_W)], roww)

      def scan(j, cnt):
        c16 = colw[pl.ds(j * L, L)]
        m = (c16 >= base) & (c16 < hi)
        r16 = roww[pl.ds(j * L, L)]
        w16 = ewb[pl.ds(lax.rem(j, 4) * L, L)]
        lc16 = c16 - init_base
        cs = plsc.cumsum(m.astype(jnp.int32))
        pos = cs + (cnt - 1)
        plsc.store_scatter(flr, [pos], r16, mask=m)
        plsc.store_scatter(flc, [pos], lc16, mask=m)
        plsc.store_scatter(flw, [pos], w16, mask=m)
        return cnt + cs[L - 1]
      cnt = lax.fori_loop(0, _HOP_W // L, scan, cnt)
      return drain(cnt, 0)

    cnt = lax.fori_loop(0, _E_PER_TILE // _HOP_W, win, 0)

    cnt_pad = jnp.where(cnt > 0, ((cnt + _GK - 1) // _GK) * _GK, 0)
    def pad(i, _):
      @pl.when(cnt + i * L < cnt_pad)
      def _():
        off = cnt + i * L
        flr[pl.ds(off, L)] = lane
        flc[pl.ds(off, L)] = lane
        flw[pl.ds(off, L)] = jnp.zeros((L,), jnp.float32)
      return 0
    lax.fori_loop(0, _GK // L, pad, 0)
    drain(cnt_pad, 0)

    plsc.subcore_barrier()
    @pl.when(jnp.logical_not(is_tail))
    def _():
      pltpu.sync_copy(acc.at[pl.ds(s * _RPT, _RPT)],
                      dst_hbm.at[pl.ds(init_base + s * _RPT, _RPT)])
    @pl.when(is_tail)
    def _():
      pltpu.sync_copy(
          acc.at[pl.ds(_R - _TAIL_ROWS + s * _TAIL_RPT, _TAIL_RPT)],
          dst_hbm.at[pl.ds(_TAIL_BASE + s * _TAIL_RPT, _TAIL_RPT)])
    plsc.subcore_barrier()
    return 0

  lax.fori_loop(0, _PASSES, do_pass, 0)


def _k_hop(src, index, w64):
  f = pl.kernel(
      _hop_body,
      out_type=jax.ShapeDtypeStruct((N_NODES, NUM_IN), jnp.float32),
      mesh=plsc.VectorSubcoreMesh(**_MESH),
      compiler_params=pltpu.CompilerParams(needs_layout_passes=False),
      scratch_types=[
          pltpu.VMEM_SHARED((_R, NUM_IN), jnp.float32),
          pltpu.VMEM((_LCAP,), jnp.int32),
          pltpu.VMEM((_LCAP,), jnp.int32),
          pltpu.VMEM((_LCAP,), jnp.float32),
          pltpu.VMEM((_GK,), jnp.int32),
          pltpu.VMEM((_GK,), jnp.int32),
          pltpu.VMEM((_GK,), jnp.float32),
          pltpu.VMEM((_GK, NUM_IN), jnp.float32),
          pltpu.VMEM((_HOP_W,), jnp.int32),
          pltpu.VMEM((_HOP_W,), jnp.int32),
          pltpu.VMEM((E_PER_GRAPH,), jnp.float32),
          pltpu.SemaphoreType.DMA,
      ],
  )
  return f(src, index, w64)


_POOL_W = 2048
_ROWS_PER_TILE = N_NODES // (NC * NS)
_G_PER_TILE = N_GRAPHS // NS
LG = 16


_PACC = N_GRAPHS * LG
_ZP = _PACC // NS


def _pool_body(lg_hbm, batch_hbm, out_hbm, pacc, zbuf, rbuf, bbuf, ibuf):
  c = lax.axis_index("c")
  s = lax.axis_index("s")
  lane = lax.iota(jnp.int32, L)

  def zero_vec(i, _):
    zbuf[pl.ds(i * L, L)] = jnp.zeros((L,), jnp.float32)
    return 0
  lax.fori_loop(0, _ZP // L, zero_vec, 0)
  pltpu.sync_copy(zbuf, pacc.at[pl.ds(s * _ZP, _ZP)])
  plsc.subcore_barrier()

  tile_base = (c * NS + s) * _ROWS_PER_TILE

  def win(w, _):
    rbase = tile_base + w * _POOL_W
    pltpu.sync_copy(lg_hbm.at[pl.ds(rbase * LG, _POOL_W * LG)], rbuf)
    pltpu.sync_copy(batch_hbm.at[pl.ds(rbase, _POOL_W)], bbuf)

    mall = lane >= 0
    def expand(g, _):
      bv = bbuf[pl.ds(g * L, L)] * LG
      ppos = g * (L * LG) + lane * LG
      for j in range(LG):
        plsc.store_scatter(ibuf, [ppos + j], bv + j, mask=mall)
      return 0
    lax.fori_loop(0, _POOL_W // L, expand, 0)
    pltpu.sync_copy(rbuf, pacc.at[ibuf], add=True)
    return 0
  lax.fori_loop(0, _ROWS_PER_TILE // _POOL_W, win, 0)
  plsc.subcore_barrier()

  pltpu.sync_copy(pacc.at[pl.ds(s * _ZP, _ZP)],
                  out_hbm.at[c, pl.ds(s * _ZP, _ZP)])


def _k_pool(lg_flat, batch):
  f = pl.kernel(
      _pool_body,
      out_type=jax.ShapeDtypeStruct((NC, _PACC), jnp.float32),
      mesh=plsc.VectorSubcoreMesh(**_MESH),
      compiler_params=pltpu.CompilerParams(needs_layout_passes=False),
      scratch_types=[
          pltpu.VMEM_SHARED((_PACC,), jnp.float32),
          pltpu.VMEM((_ZP,), jnp.float32),
          pltpu.VMEM((_POOL_W * LG,), jnp.float32),
          pltpu.VMEM((_POOL_W,), jnp.int32),
          pltpu.VMEM((_POOL_W * LG,), jnp.int32),
      ],
  )
  return f(lg_flat, batch)


_PRE_B = 2048
_NR = N_NODES // NUM_IN
_PRE_R = _PRE_B // NUM_IN


def _pre_body(degp_ref, x_ref, dinv_ref, dinv2_ref, y0_ref):
  deg = 1.0 + degp_ref[0] + degp_ref[1]
  dinv = lax.rsqrt(deg)
  dinv_ref[...] = dinv
  dinv2_ref[...] = 1.0 / deg
  y0_ref[...] = x_ref[...] * dinv[:, :, None]


def _k_pre(degp, x):
  grid = N_NODES // _PRE_B
  return pl.pallas_call(
      _pre_body,
      grid=(grid,),
      in_specs=[
          pl.BlockSpec((NC, _PRE_R, NUM_IN), lambda i: (0, i, 0)),
          pl.BlockSpec((_PRE_R, NUM_IN, NUM_IN), lambda i: (i, 0, 0)),
      ],
      out_specs=[
          pl.BlockSpec((_PRE_R, NUM_IN), lambda i: (i, 0)),
          pl.BlockSpec((_PRE_R, NUM_IN), lambda i: (i, 0)),
          pl.BlockSpec((_PRE_R, NUM_IN, NUM_IN), lambda i: (i, 0, 0)),
      ],
      out_shape=[
          jax.ShapeDtypeStruct((_NR, NUM_IN), jnp.float32),
          jax.ShapeDtypeStruct((_NR, NUM_IN), jnp.float32),
          jax.ShapeDtypeStruct((_NR, NUM_IN, NUM_IN), jnp.float32),
      ],
  )(degp.reshape(NC, _NR, NUM_IN), x.reshape(_NR, NUM_IN, NUM_IN))


def _scale_body(y_ref, d_ref, o_ref):
  o_ref[...] = y_ref[...] * d_ref[...][:, :, None]


def _k_scale(y3d, d2d):
  grid = _NR // _PRE_R
  return pl.pallas_call(
      _scale_body,
      grid=(grid,),
      in_specs=[
          pl.BlockSpec((_PRE_R, NUM_IN, NUM_IN), lambda i: (i, 0, 0)),
          pl.BlockSpec((_PRE_R, NUM_IN), lambda i: (i, 0)),
      ],
      out_specs=pl.BlockSpec((_PRE_R, NUM_IN, NUM_IN), lambda i: (i, 0, 0)),
      out_shape=jax.ShapeDtypeStruct((_NR, NUM_IN, NUM_IN), jnp.float32),
  )(y3d, d2d)


_MM_B = 2048
_MM_R = _MM_B // NUM_IN


def _mm_body(y3_ref, dinv_ref, linwT_ref, linb_ref, fcwT_ref, o_ref):
  h2 = (y3_ref[...] * dinv_ref[...][:, :, None]).reshape(_MM_B, NUM_IN)
  h = jnp.dot(h2, linwT_ref[...], preferred_element_type=jnp.float32)
  h = jnp.maximum(h + linb_ref[...], 0.0)
  o_ref[...] = jnp.dot(h, fcwT_ref[...], preferred_element_type=jnp.float32)


def _k_mm(y3_3d, dinv2d, linwT, linb, fcwT):
  grid = N_NODES // _MM_B
  return pl.pallas_call(
      _mm_body,
      grid=(grid,),
      in_specs=[
          pl.BlockSpec((_MM_R, NUM_IN, NUM_IN), lambda i: (i, 0, 0)),
          pl.BlockSpec((_MM_R, NUM_IN), lambda i: (i, 0)),
          pl.BlockSpec((NUM_IN, NUM_HIDDEN), lambda i: (0, 0)),
          pl.BlockSpec((1, NUM_HIDDEN), lambda i: (0, 0)),
          pl.BlockSpec((NUM_HIDDEN, LG), lambda i: (0, 0)),
      ],
      out_specs=pl.BlockSpec((_MM_B, LG), lambda i: (i, 0)),
      out_shape=jax.ShapeDtypeStruct((N_NODES, LG), jnp.float32),
  )(y3_3d, dinv2d, linwT, linb, fcwT)


def _soft_body(pp_ref, fcb_ref, o_ref):
  z = pp_ref[0] + pp_ref[1] + fcb_ref[...]
  col = lax.broadcasted_iota(jnp.int32, (N_GRAPHS, LG), 1)
  valid = col < NUM_CLASS
  z = jnp.where(valid, z, -1e30)
  z = z - jnp.max(z, axis=1, keepdims=True)
  p = jnp.exp(z)
  p = jnp.where(valid, p, 0.0)
  o_ref[...] = p / jnp.sum(p, axis=1, keepdims=True)


def _k_soft(pooledp, fcb):
  return pl.pallas_call(
      _soft_body,
      in_specs=[
          pl.BlockSpec((NC, N_GRAPHS, LG), lambda: (0, 0, 0)),
          pl.BlockSpec((1, LG), lambda: (0, 0)),
      ],
      out_specs=pl.BlockSpec((N_GRAPHS, LG), lambda: (0, 0)),
      out_shape=jax.ShapeDtypeStruct((N_GRAPHS, LG), jnp.float32),
  )(pooledp, fcb)


def kernel(x, index, batch, weight, lin_w, lin_b, fc_w, fc_b):
  index = index.astype(jnp.int32)
  batch = batch.astype(jnp.int32)
  ew_win = jnp.tile(weight, _DEG_W // E_PER_GRAPH)

  degp = _k_deg(index, ew_win)
  dinv, dinv2, y0_3d = _k_pre(degp, x)
  y1 = _k_hop(y0_3d.reshape(N_NODES, NUM_IN), index, weight)
  y2_3d = _k_scale(y1.reshape(_NR, NUM_IN, NUM_IN), dinv2)
  y3 = _k_hop(y2_3d.reshape(N_NODES, NUM_IN), index, weight)

  linwT = lin_w.T
  linb = lin_b.reshape(1, NUM_HIDDEN)
  fcwT = jnp.zeros((NUM_HIDDEN, LG), jnp.float32).at[:, :NUM_CLASS].set(fc_w.T)
  lg = _k_mm(y3.reshape(_NR, NUM_IN, NUM_IN), dinv, linwT, linb, fcwT)

  pooledp = _k_pool(lg.reshape(-1), batch)
  fcb = jnp.zeros((1, LG), jnp.float32).at[0, :NUM_CLASS].set(fc_b)
  probs = _k_soft(pooledp.reshape(NC, N_GRAPHS, LG), fcb)
  return probs[:, :NUM_CLASS]
